# baseline (speedup 1.0000x reference)
import jax
import jax.numpy as jnp
from jax.experimental import pallas as pl
from jax.experimental.pallas import tpu as pltpu

_N, _C, _H, _W = 8, 19, 512, 512
_HW = _H * _W
_LANES = 128
_ROWS = _HW // _LANES
_TR = 512
_K = _ROWS // _TR
_CPAD = 24


def _main_body(x_ref, hist_ref, ssq_ref, e_ref):
    k = pl.program_id(1)

    @pl.when(k == 0)
    def _init():
        hist_ref[...] = jnp.zeros_like(hist_ref)
        ssq_ref[...] = jnp.zeros_like(ssq_ref)

    x = x_ref[0]

    m = x[0]
    idx = jnp.zeros(m.shape, jnp.int32)
    for c in range(1, _C):
        gt = x[c] > m
        m = jnp.where(gt, x[c], m)
        idx = jnp.where(gt, c, idx)

    s = jnp.zeros_like(m)
    for c in range(_C):
        e = jnp.exp(x[c] - m)
        e_ref[c] = e
        s = s + e
    r2 = 1.0 / (s * s)

    zpad = jnp.zeros((_CPAD - _C, _LANES), jnp.float32)
    hrows = [
        jnp.sum(jnp.where(idx == c, 1.0, 0.0), axis=0, keepdims=True)
        for c in range(_C)
    ]
    qrows = []
    for c in range(_C):
        e = e_ref[c]
        qrows.append(jnp.sum(e * e * r2, axis=0, keepdims=True))
    hist_blk = jnp.concatenate(hrows + [zpad], axis=0)
    ssq_blk = jnp.concatenate(qrows + [zpad], axis=0)
    hist_ref[...] = hist_ref[...] + hist_blk[None]
    ssq_ref[...] = ssq_ref[...] + ssq_blk[None]


def _combine_body(h_ref, q_ref, o_ref):
    h = jnp.sum(h_ref[...], axis=2)
    q = jnp.sum(q_ref[...], axis=2)
    col = jax.lax.broadcasted_iota(jnp.int32, (_N, _CPAD), 1)
    mask = col < _C
    hadj = jnp.where(h == 0.0, 1.0, h)
    total = jnp.sum(jnp.where(mask, hadj, 0.0), axis=1, keepdims=True)
    w = jnp.exp(0.2 * (jnp.log(total) - jnp.log(hadj)))
    loss = -jnp.sum(jnp.where(mask, w * q, 0.0))
    o_ref[0, 0] = loss * (1.0 / (_N * _C * _H * _W))


def kernel(inputs):
    x = inputs.reshape(_N, _C, _ROWS, _LANES)
    hist, ssq = pl.pallas_call(
        _main_body,
        grid=(_N, _K),
        in_specs=[
            pl.BlockSpec((1, _C, _TR, _LANES), lambda n, k: (n, 0, k, 0)),
        ],
        out_specs=[
            pl.BlockSpec((1, _CPAD, _LANES), lambda n, k: (n, 0, 0)),
            pl.BlockSpec((1, _CPAD, _LANES), lambda n, k: (n, 0, 0)),
        ],
        out_shape=[
            jax.ShapeDtypeStruct((_N, _CPAD, _LANES), jnp.float32),
            jax.ShapeDtypeStruct((_N, _CPAD, _LANES), jnp.float32),
        ],
        scratch_shapes=[pltpu.VMEM((_C, _TR, _LANES), jnp.float32)],
        compiler_params=pltpu.CompilerParams(
            dimension_semantics=("parallel", "arbitrary"),
        ),
    )(x)

    out = pl.pallas_call(
        _combine_body,
        out_shape=jax.ShapeDtypeStruct((1, 1), jnp.float32),
        out_specs=pl.BlockSpec(memory_space=pltpu.SMEM),
    )(hist, ssq)
    return out[0, 0]


# speedup vs baseline: 1.0849x; 1.0849x over previous
import jax
import jax.numpy as jnp
from jax.experimental import pallas as pl
from jax.experimental.pallas import tpu as pltpu

_N, _C, _H, _W = 8, 19, 512, 512
_HW = _H * _W
_LANES = 128
_ROWS = _HW // _LANES
_TR = 1024
_K = _ROWS // _TR
_CPAD = 24


def _main_body(x_ref, hist_ref, ssq_ref, e_ref):
    k = pl.program_id(1)

    @pl.when(k == 0)
    def _init():
        hist_ref[...] = jnp.zeros_like(hist_ref)
        ssq_ref[...] = jnp.zeros_like(ssq_ref)

    x = x_ref[0]

    m = x[0]
    for c in range(1, _C):
        m = jnp.maximum(m, x[c])

    s = jnp.zeros_like(m)
    hrows = []
    for c in range(_C):
        e = jnp.exp(x[c] - m)
        e_ref[c] = e
        s = s + e
        hrows.append(jnp.sum(jnp.where(e == 1.0, 1.0, 0.0), axis=0, keepdims=True))
    r2 = 1.0 / (s * s)

    qrows = []
    for c in range(_C):
        e = e_ref[c]
        qrows.append(jnp.sum(e * e * r2, axis=0, keepdims=True))

    zpad = jnp.zeros((_CPAD - _C, _LANES), jnp.float32)
    hist_blk = jnp.concatenate(hrows + [zpad], axis=0)
    ssq_blk = jnp.concatenate(qrows + [zpad], axis=0)
    hist_ref[...] = hist_ref[...] + hist_blk[None]
    ssq_ref[...] = ssq_ref[...] + ssq_blk[None]


def _combine_body(h_ref, q_ref, o_ref):
    h = jnp.sum(h_ref[...], axis=2)
    q = jnp.sum(q_ref[...], axis=2)
    col = jax.lax.broadcasted_iota(jnp.int32, (_N, _CPAD), 1)
    mask = col < _C
    hadj = jnp.where(h == 0.0, 1.0, h)
    total = jnp.sum(jnp.where(mask, hadj, 0.0), axis=1, keepdims=True)
    w = jnp.exp(0.2 * (jnp.log(total) - jnp.log(hadj)))
    loss = -jnp.sum(jnp.where(mask, w * q, 0.0))
    o_ref[0, 0] = loss * (1.0 / (_N * _C * _H * _W))


def kernel(inputs):
    x = inputs.reshape(_N, _C, _ROWS, _LANES)
    hist, ssq = pl.pallas_call(
        _main_body,
        grid=(_N, _K),
        in_specs=[
            pl.BlockSpec((1, _C, _TR, _LANES), lambda n, k: (n, 0, k, 0)),
        ],
        out_specs=[
            pl.BlockSpec((1, _CPAD, _LANES), lambda n, k: (n, 0, 0)),
            pl.BlockSpec((1, _CPAD, _LANES), lambda n, k: (n, 0, 0)),
        ],
        out_shape=[
            jax.ShapeDtypeStruct((_N, _CPAD, _LANES), jnp.float32),
            jax.ShapeDtypeStruct((_N, _CPAD, _LANES), jnp.float32),
        ],
        scratch_shapes=[pltpu.VMEM((_C, _TR, _LANES), jnp.float32)],
        compiler_params=pltpu.CompilerParams(
            dimension_semantics=("parallel", "arbitrary"),
        ),
    )(x)

    out = pl.pallas_call(
        _combine_body,
        out_shape=jax.ShapeDtypeStruct((1, 1), jnp.float32),
        out_specs=pl.BlockSpec(memory_space=pltpu.SMEM),
    )(hist, ssq)
    return out[0, 0]
